# baseline (speedup 1.0000x reference)
import functools

import jax
import jax.numpy as jnp
from jax import lax
from jax.experimental import pallas as pl
from jax.experimental.pallas import tpu as pltpu
from jax.experimental.pallas import tpu_sc as plsc

_LAMBDA = 0.2
_E = 160000
_D = 256
_NW = 32
_EPW = 5120
_K = 64
_NCHUNK = _EPW // _K
_E_PAD = _NW * _EPW


def _rsqrt(p):
    i = lax.bitcast_convert_type(p, jnp.int32)
    i = jnp.int32(0x5F3759DF) - (i >> 1)
    y = lax.bitcast_convert_type(i, jnp.float32)
    y = y * (1.5 - 0.5 * p * y * y)
    y = y * (1.5 - 0.5 * p * y * y)
    y = y * (1.5 - 0.5 * p * y * y)
    return y


def _build():
    mesh = plsc.VectorSubcoreMesh(core_axis_name="c", subcore_axis_name="s")

    row_scratch = []
    for _ in range(4):
        row_scratch += [
            pltpu.VMEM((_K,), jnp.int32),
            pltpu.VMEM((_K,), jnp.int32),
            pltpu.VMEM((_K, _D // 2), jnp.float32),
            pltpu.VMEM((_K, _D // 2), jnp.float32),
            pltpu.SemaphoreType.DMA,
        ]

    @functools.partial(
        pl.kernel,
        out_type=jax.ShapeDtypeStruct((_NW, 32), jnp.float32),
        mesh=mesh,
        compiler_params=pltpu.CompilerParams(needs_layout_passes=False),
        scratch_types=row_scratch + [pltpu.VMEM((32,), jnp.float32)],
    )
    def k(table, sidx_hbm, tidx_hbm, out, *scratch):
        bufs = tuple(tuple(scratch[5 * b:5 * b + 5]) for b in range(4))
        outbuf = scratch[20]
        wid = lax.axis_index("s") * 2 + lax.axis_index("c")
        base_w = wid * _EPW

        def fire(jb, b):
            sidx, tidx, srows, trows, sem = bufs[b]
            base = base_w + jb * _K
            pltpu.sync_copy(sidx_hbm.at[pl.ds(base, _K)], sidx)
            pltpu.sync_copy(tidx_hbm.at[pl.ds(base, _K)], tidx)
            pltpu.async_copy(table.at[sidx], srows, sem)
            pltpu.async_copy(table.at[tidx], trows, sem)

        def drain(b):
            sidx, tidx, srows, trows, sem = bufs[b]
            pltpu.make_async_copy(table.at[sidx], srows, sem).wait()
            pltpu.make_async_copy(table.at[tidx], trows, sem).wait()

        def compute(jb, b, acc):
            _, _, srows, trows, _ = bufs[b]
            base = base_w + jb * _K

            def edge_body(i, carry):
                a_cos, a_sq = carry
                d = ns = nt = None
                for c in range(8):
                    sa, sb = plsc.unpack(
                        plsc.bitcast(srows[i, pl.ds(c * 16, 16)], jnp.bfloat16),
                        format=plsc.PackFormat.INTERLEAVED,
                        preferred_element_type=jnp.float32)
                    ta, tb = plsc.unpack(
                        plsc.bitcast(trows[i, pl.ds(c * 16, 16)], jnp.bfloat16),
                        format=plsc.PackFormat.INTERLEAVED,
                        preferred_element_type=jnp.float32)
                    if d is None:
                        d = sa * ta + sb * tb
                        ns = sa * sa + sb * sb
                        nt = ta * ta + tb * tb
                    else:
                        d = d + sa * ta + sb * tb
                        ns = ns + sa * sa + sb * sb
                        nt = nt + ta * ta + tb * tb
                dsc = jnp.sum(d)
                nssc = jnp.sum(ns)
                ntsc = jnp.sum(nt)
                p = jnp.maximum(nssc, 1e-16) * jnp.maximum(ntsc, 1e-16)
                cos = dsc * _rsqrt(p)
                valid = (base + i) < _E
                a_cos = a_cos + jnp.where(valid, 1.0 - cos, 0.0)
                a_sq = a_sq + jnp.where(valid, nssc + ntsc - 2.0 * dsc, 0.0)
                return (a_cos, a_sq)

            return plsc.parallel_loop(0, _K, 1, unroll=4, carry=acc)(edge_body)

        fire(0, 0)
        fire(1, 1)
        fire(2, 2)

        def group_body(j4, acc):
            for b in range(4):
                c = j4 * 4 + b
                drain(b)
                acc = compute(c, b, acc)

                @pl.when(c + 3 < _NCHUNK)
                def _():
                    fire(c + 3, (b + 3) % 4)

            return acc

        acc_cos, acc_sq = lax.fori_loop(
            0, _NCHUNK // 4, group_body,
            (jnp.float32(0.0), jnp.float32(0.0)))
        lanes = lax.iota(jnp.int32, 16)
        first = lanes < 1
        outbuf[pl.ds(0, 16)] = jnp.where(
            first, acc_cos * (1.0 / _E), 0.0)
        outbuf[pl.ds(16, 16)] = jnp.where(
            first, acc_sq * (_LAMBDA / (_E * _D)), 0.0)
        pltpu.sync_copy(outbuf, out.at[wid])

    return k


_sc_kernel = _build()


def kernel(embedding, edge_index):
    ei = edge_index.astype(jnp.int32)
    pad = _E_PAD - _E
    src = jnp.concatenate([ei[0], jnp.zeros((pad,), jnp.int32)])
    tgt = jnp.concatenate([ei[1], jnp.zeros((pad,), jnp.int32)])
    lo = lax.bitcast_convert_type(
        embedding[:, :_D // 2].astype(jnp.bfloat16), jnp.uint16)
    hi = lax.bitcast_convert_type(
        embedding[:, _D // 2:].astype(jnp.bfloat16), jnp.uint16)
    table = lax.bitcast_convert_type(
        lo.astype(jnp.uint32) | (hi.astype(jnp.uint32) << 16), jnp.float32)
    parts = _sc_kernel(table, src, tgt)
    return jnp.sum(parts)


# speedup vs baseline: 1.4397x; 1.4397x over previous
import functools

import jax
import jax.numpy as jnp
from jax import lax
from jax.experimental import pallas as pl
from jax.experimental.pallas import tpu as pltpu
from jax.experimental.pallas import tpu_sc as plsc

_LAMBDA = 0.2
_E = 160000
_D = 256
_NW = 32
_EPW = 5040
_K = 120
_NCHUNK = _EPW // _K
_E_PAD = _NW * _EPW


def _rsqrt(p):
    i = lax.bitcast_convert_type(p, jnp.int32)
    i = jnp.int32(0x5F3759DF) - (i >> 1)
    y = lax.bitcast_convert_type(i, jnp.float32)
    y = y * (1.5 - 0.5 * p * y * y)
    y = y * (1.5 - 0.5 * p * y * y)
    y = y * (1.5 - 0.5 * p * y * y)
    return y


def _build():
    mesh = plsc.VectorSubcoreMesh(core_axis_name="c", subcore_axis_name="s")

    row_scratch = []
    for _ in range(3):
        row_scratch += [
            pltpu.VMEM((_K,), jnp.int32),
            pltpu.VMEM((_K,), jnp.int32),
            pltpu.VMEM((_K, _D // 2), jnp.float32),
            pltpu.VMEM((_K, _D // 2), jnp.float32),
            pltpu.SemaphoreType.DMA,
        ]

    @functools.partial(
        pl.kernel,
        out_type=jax.ShapeDtypeStruct((_NW, 32), jnp.float32),
        mesh=mesh,
        compiler_params=pltpu.CompilerParams(needs_layout_passes=False),
        scratch_types=row_scratch + [pltpu.VMEM((32,), jnp.float32)],
    )
    def k(table, sidx_hbm, tidx_hbm, out, *scratch):
        bufs = tuple(tuple(scratch[5 * b:5 * b + 5]) for b in range(3))
        outbuf = scratch[15]
        wid = lax.axis_index("s") * 2 + lax.axis_index("c")
        base_w = wid * _EPW

        def fire(jb, b):
            sidx, tidx, srows, trows, sem = bufs[b]
            base = base_w + jb * _K
            pltpu.sync_copy(sidx_hbm.at[pl.ds(base, _K)], sidx)
            pltpu.sync_copy(tidx_hbm.at[pl.ds(base, _K)], tidx)
            pltpu.async_copy(table.at[sidx], srows, sem)
            pltpu.async_copy(table.at[tidx], trows, sem)

        def drain(b):
            sidx, tidx, srows, trows, sem = bufs[b]
            pltpu.make_async_copy(table.at[sidx], srows, sem).wait()
            pltpu.make_async_copy(table.at[tidx], trows, sem).wait()

        def compute(jb, b, acc):
            _, _, srows, trows, _ = bufs[b]
            base = base_w + jb * _K

            def edge_body(i, carry):
                a_cos, a_sq = carry
                d = ns = nt = None
                for c in range(8):
                    sa, sb = plsc.unpack(
                        plsc.bitcast(srows[i, pl.ds(c * 16, 16)], jnp.bfloat16),
                        format=plsc.PackFormat.INTERLEAVED,
                        preferred_element_type=jnp.float32)
                    ta, tb = plsc.unpack(
                        plsc.bitcast(trows[i, pl.ds(c * 16, 16)], jnp.bfloat16),
                        format=plsc.PackFormat.INTERLEAVED,
                        preferred_element_type=jnp.float32)
                    if d is None:
                        d = sa * ta + sb * tb
                        ns = sa * sa + sb * sb
                        nt = ta * ta + tb * tb
                    else:
                        d = d + sa * ta + sb * tb
                        ns = ns + sa * sa + sb * sb
                        nt = nt + ta * ta + tb * tb
                dsc = jnp.sum(d)
                nssc = jnp.sum(ns)
                ntsc = jnp.sum(nt)
                p = jnp.maximum(nssc, 1e-16) * jnp.maximum(ntsc, 1e-16)
                cos = dsc * _rsqrt(p)
                valid = (base + i) < _E
                a_cos = a_cos + jnp.where(valid, 1.0 - cos, 0.0)
                a_sq = a_sq + jnp.where(valid, nssc + ntsc - 2.0 * dsc, 0.0)
                return (a_cos, a_sq)

            return plsc.parallel_loop(0, _K, 1, unroll=4, carry=acc)(edge_body)

        fire(0, 0)
        fire(1, 1)

        def group_body(j3, acc):
            for b in range(3):
                c = j3 * 3 + b
                drain(b)
                acc = compute(c, b, acc)

                @pl.when(c + 2 < _NCHUNK)
                def _():
                    fire(c + 2, (b + 2) % 3)

            return acc

        acc_cos, acc_sq = lax.fori_loop(
            0, _NCHUNK // 3, group_body,
            (jnp.float32(0.0), jnp.float32(0.0)))
        lanes = lax.iota(jnp.int32, 16)
        first = lanes < 1
        outbuf[pl.ds(0, 16)] = jnp.where(
            first, acc_cos * (1.0 / _E), 0.0)
        outbuf[pl.ds(16, 16)] = jnp.where(
            first, acc_sq * (_LAMBDA / (_E * _D)), 0.0)
        pltpu.sync_copy(outbuf, out.at[wid])

    return k


_sc_kernel = _build()


def kernel(embedding, edge_index):
    ei = edge_index.astype(jnp.int32)
    pad = _E_PAD - _E
    src = jnp.concatenate([ei[0], jnp.zeros((pad,), jnp.int32)])
    tgt = jnp.concatenate([ei[1], jnp.zeros((pad,), jnp.int32)])
    lo = lax.bitcast_convert_type(
        embedding[:, :_D // 2].astype(jnp.bfloat16), jnp.uint16)
    hi = lax.bitcast_convert_type(
        embedding[:, _D // 2:].astype(jnp.bfloat16), jnp.uint16)
    table = lax.bitcast_convert_type(
        lo.astype(jnp.uint32) | (hi.astype(jnp.uint32) << 16), jnp.float32)
    parts = _sc_kernel(table, src, tgt)
    return jnp.sum(parts)


# speedup vs baseline: 1.5128x; 1.0508x over previous
import functools

import jax
import jax.numpy as jnp
from jax import lax
from jax.experimental import pallas as pl
from jax.experimental.pallas import tpu as pltpu
from jax.experimental.pallas import tpu_sc as plsc

_LAMBDA = 0.2
_E = 160000
_D = 256
_NW = 32
_K = 120
_NCHUNK0 = 45
_NCHUNK1 = 39
_EPW0 = _NCHUNK0 * _K
_EPW1 = _NCHUNK1 * _K
_EPS = _EPW0 + _EPW1
_E_PAD = 16 * _EPS


def _rsqrt(p):
    i = lax.bitcast_convert_type(p, jnp.int32)
    i = jnp.int32(0x5F3759DF) - (i >> 1)
    y = lax.bitcast_convert_type(i, jnp.float32)
    y = y * (1.5 - 0.5 * p * y * y)
    y = y * (1.5 - 0.5 * p * y * y)
    y = y * (1.5 - 0.5 * p * y * y)
    return y


def _build():
    mesh = plsc.VectorSubcoreMesh(core_axis_name="c", subcore_axis_name="s")

    row_scratch = []
    for _ in range(3):
        row_scratch += [
            pltpu.VMEM((_K,), jnp.int32),
            pltpu.VMEM((_K,), jnp.int32),
            pltpu.VMEM((_K, _D // 2), jnp.float32),
            pltpu.VMEM((_K, _D // 2), jnp.float32),
            pltpu.SemaphoreType.DMA,
        ]

    @functools.partial(
        pl.kernel,
        out_type=jax.ShapeDtypeStruct((_NW, 32), jnp.float32),
        mesh=mesh,
        compiler_params=pltpu.CompilerParams(needs_layout_passes=False),
        scratch_types=row_scratch + [pltpu.VMEM((32,), jnp.float32)],
    )
    def k(table, sidx_hbm, tidx_hbm, out, *scratch):
        bufs = tuple(tuple(scratch[5 * b:5 * b + 5]) for b in range(3))
        outbuf = scratch[15]
        cid = lax.axis_index("c")
        sid = lax.axis_index("s")
        wid = sid * 2 + cid
        base_w = sid * _EPS + cid * _EPW0
        nchunk = jnp.where(cid == 0, _NCHUNK0, _NCHUNK1)
        ngroup = jnp.where(cid == 0, _NCHUNK0 // 3, _NCHUNK1 // 3)

        def fire(jb, b):
            sidx, tidx, srows, trows, sem = bufs[b]
            base = base_w + jb * _K
            pltpu.sync_copy(sidx_hbm.at[pl.ds(base, _K)], sidx)
            pltpu.sync_copy(tidx_hbm.at[pl.ds(base, _K)], tidx)
            pltpu.async_copy(table.at[sidx], srows, sem)
            pltpu.async_copy(table.at[tidx], trows, sem)

        def drain(b):
            sidx, tidx, srows, trows, sem = bufs[b]
            pltpu.make_async_copy(table.at[sidx], srows, sem).wait()
            pltpu.make_async_copy(table.at[tidx], trows, sem).wait()

        def compute(jb, b, acc):
            _, _, srows, trows, _ = bufs[b]
            base = base_w + jb * _K

            def edge_body(i, carry):
                a_cos, a_sq = carry
                d = ns = nt = None
                for c in range(8):
                    sa, sb = plsc.unpack(
                        plsc.bitcast(srows[i, pl.ds(c * 16, 16)], jnp.bfloat16),
                        format=plsc.PackFormat.INTERLEAVED,
                        preferred_element_type=jnp.float32)
                    ta, tb = plsc.unpack(
                        plsc.bitcast(trows[i, pl.ds(c * 16, 16)], jnp.bfloat16),
                        format=plsc.PackFormat.INTERLEAVED,
                        preferred_element_type=jnp.float32)
                    if d is None:
                        d = sa * ta + sb * tb
                        ns = sa * sa + sb * sb
                        nt = ta * ta + tb * tb
                    else:
                        d = d + sa * ta + sb * tb
                        ns = ns + sa * sa + sb * sb
                        nt = nt + ta * ta + tb * tb
                dsc = jnp.sum(d)
                nssc = jnp.sum(ns)
                ntsc = jnp.sum(nt)
                p = jnp.maximum(nssc, 1e-16) * jnp.maximum(ntsc, 1e-16)
                cos = dsc * _rsqrt(p)
                valid = (base + i) < _E
                a_cos = a_cos + jnp.where(valid, 1.0 - cos, 0.0)
                a_sq = a_sq + jnp.where(valid, nssc + ntsc - 2.0 * dsc, 0.0)
                return (a_cos, a_sq)

            return plsc.parallel_loop(0, _K, 1, unroll=4, carry=acc)(edge_body)

        fire(0, 0)
        fire(1, 1)

        def group_body(j3, acc):
            for b in range(3):
                c = j3 * 3 + b
                drain(b)
                acc = compute(c, b, acc)

                @pl.when(c + 2 < nchunk)
                def _():
                    fire(c + 2, (b + 2) % 3)

            return acc

        acc_cos, acc_sq = lax.fori_loop(
            0, ngroup, group_body,
            (jnp.float32(0.0), jnp.float32(0.0)))
        lanes = lax.iota(jnp.int32, 16)
        first = lanes < 1
        outbuf[pl.ds(0, 16)] = jnp.where(
            first, acc_cos * (1.0 / _E), 0.0)
        outbuf[pl.ds(16, 16)] = jnp.where(
            first, acc_sq * (_LAMBDA / (_E * _D)), 0.0)
        pltpu.sync_copy(outbuf, out.at[wid])

    return k


_sc_kernel = _build()


def kernel(embedding, edge_index):
    ei = edge_index.astype(jnp.int32)
    pad = _E_PAD - _E
    src = jnp.concatenate([ei[0], jnp.zeros((pad,), jnp.int32)])
    tgt = jnp.concatenate([ei[1], jnp.zeros((pad,), jnp.int32)])
    lo = lax.bitcast_convert_type(
        embedding[:, :_D // 2].astype(jnp.bfloat16), jnp.uint16)
    hi = lax.bitcast_convert_type(
        embedding[:, _D // 2:].astype(jnp.bfloat16), jnp.uint16)
    table = lax.bitcast_convert_type(
        lo.astype(jnp.uint32) | (hi.astype(jnp.uint32) << 16), jnp.float32)
    parts = _sc_kernel(table, src, tgt)
    return jnp.sum(parts)
